# baseline (speedup 1.0000x reference)
import functools

import jax
import jax.numpy as jnp
from jax import lax
from jax.experimental import pallas as pl
from jax.experimental.pallas import tpu as pltpu
from jax.experimental.pallas import tpu_sc as plsc

B = 16384
L = 20
D = 32
NC = 2
NS = 16
NW = NC * NS
PER_W = B // NW
C = 128
NCHUNK = PER_W // C
TOKR = (C * L) // 128
LANES = 16

_COMPILER_PARAMS = pltpu.CompilerParams(use_tc_tiling_on_sc=False,
                                        needs_layout_passes=False)
_MESH = plsc.VectorSubcoreMesh(core_axis_name="c", subcore_axis_name="s")


CH = 64
NCH = PER_W // CH
CTOKR = (CH * L) // 128


def _text_body(tokr_hbm, text_hbm, out_hbm,
               tokidx_v, rows0_v, rows1_v, out0_v, out1_v, scale_v, row0_v,
               sem0, sem1, osem0, osem1):
    rows_v = (rows0_v, rows1_v)
    out_v = (out0_v, out1_v)
    sems = (sem0, sem1)
    osems = (osem0, osem1)
    wid = lax.axis_index("s") * NC + lax.axis_index("c")
    iota = lax.iota(jnp.int32, LANES)
    zv = jnp.zeros((LANES,), jnp.int32)
    ov = zv + 1

    wtok = PER_W * L
    pltpu.sync_copy(tokr_hbm.at[pl.ds(wid * wtok, wtok)], tokidx_v)
    pltpu.sync_copy(text_hbm.at[pl.ds(0, 8)], row0_v)

    def fire(c, buf):
        cps = []
        for r in range(CTOKR):
            cps.append(pltpu.async_copy(
                text_hbm.at[tokidx_v.at[pl.ds((c * CTOKR + r) * 128, 128)]],
                rows_v[buf].at[pl.ds(r * 128, 128)], sems[buf]))
        return cps

    r0a = row0_v[0, pl.ds(0, LANES)]
    r0b = row0_v[0, pl.ds(LANES, LANES)]
    inflight = {0: fire(0, 0)}
    owaits = {}

    for c in range(NCH):
        buf = c & 1
        base = wid * PER_W + c * CH

        if c + 1 < NCH:
            inflight[c + 1] = fire(c + 1, (c + 1) & 1)

        def zgroup(g, carry):
            brow = g * LANES + iota
            fb = brow * L + (c * CH * L)
            z = jnp.zeros((LANES,), jnp.float32)
            for l in range(L):
                tv = plsc.load_gather(tokidx_v, [fb + l])
                z = z + jnp.where(tv == 0, 1.0, 0.0)
            inv = 1.0 / jnp.maximum(L * 1.0 - z, 1.0)
            b0 = g * LANES
            scale_v[0, pl.ds(b0, LANES)] = inv
            scale_v[1, pl.ds(b0, LANES)] = z * inv
            return carry

        lax.fori_loop(0, CH // LANES, zgroup, 0)

        for cp in inflight.pop(c):
            cp.wait()
        if c - 2 in owaits:
            owaits.pop(c - 2).wait()

        def bloop(b, carry):
            fb = b * L
            a0 = rows_v[buf][fb, pl.ds(0, LANES)]
            a1 = rows_v[buf][fb, pl.ds(LANES, LANES)]
            for l in range(1, L):
                a0 = a0 + rows_v[buf][fb + l, pl.ds(0, LANES)]
                a1 = a1 + rows_v[buf][fb + l, pl.ds(LANES, LANES)]
            bcol = zv + b
            inv = plsc.load_gather(scale_v, [zv, bcol])
            zfi = plsc.load_gather(scale_v, [ov, bcol])
            out_v[buf][b, pl.ds(0, LANES)] = a0 * inv - zfi * r0a
            out_v[buf][b, pl.ds(LANES, LANES)] = a1 * inv - zfi * r0b
            return carry

        lax.fori_loop(0, CH, bloop, 0)
        owaits[c] = pltpu.async_copy(
            out_v[buf], out_hbm.at[pl.ds(base, CH)], osems[buf])
    for w in owaits.values():
        w.wait()


def _id_body(title_hbm, idt_hbm, text_hbm, out_hbm, ididx0_v, ididx1_v,
             idrows0_v, idrows1_v, textc0_v, textc1_v, out640_v, out641_v,
             sem0, sem1, tsem0, tsem1, osem0, osem1):
    ididx_v = (ididx0_v, ididx1_v)
    idrows_v = (idrows0_v, idrows1_v)
    textc_v = (textc0_v, textc1_v)
    out64_v = (out640_v, out641_v)
    sems = (sem0, sem1)
    tsems = (tsem0, tsem1)
    osems = (osem0, osem1)
    wid = lax.axis_index("s") * NC + lax.axis_index("c")

    def fire(c, buf):
        base = wid * PER_W + c * CH
        pltpu.sync_copy(title_hbm.at[pl.ds(base, CH)], ididx_v[buf])
        return (pltpu.async_copy(idt_hbm.at[ididx_v[buf]], idrows_v[buf],
                                 sems[buf]),
                pltpu.async_copy(text_hbm.at[pl.ds(base, CH)], textc_v[buf],
                                 tsems[buf]))

    inflight = {0: fire(0, 0)}
    owaits = {}
    for c in range(NCH):
        buf = c & 1
        base = wid * PER_W + c * CH
        if c + 1 < NCH:
            if c - 1 in owaits:
                owaits.pop(c - 1).wait()
            inflight[c + 1] = fire(c + 1, (c + 1) & 1)
        for cp in inflight.pop(c):
            cp.wait()

        def bloop(b, carry):
            out64_v[buf][b, pl.ds(0, LANES)] = idrows_v[buf][b, pl.ds(0, LANES)]
            out64_v[buf][b, pl.ds(LANES, LANES)] = (
                idrows_v[buf][b, pl.ds(LANES, LANES)])
            out64_v[buf][b, pl.ds(2 * LANES, LANES)] = (
                textc_v[buf][b, pl.ds(0, LANES)])
            out64_v[buf][b, pl.ds(3 * LANES, LANES)] = (
                textc_v[buf][b, pl.ds(LANES, LANES)])
            return carry

        lax.fori_loop(0, CH, bloop, 0)
        owaits[c] = pltpu.async_copy(
            out64_v[buf], out_hbm.at[pl.ds(base, CH)], osems[buf])
    for w in owaits.values():
        w.wait()


_text_call = functools.partial(
    pl.kernel,
    mesh=_MESH,
    compiler_params=_COMPILER_PARAMS,
    out_type=jax.ShapeDtypeStruct((B, D), jnp.float32),
    scratch_types=[
        pltpu.VMEM((PER_W * L,), jnp.int32),
        pltpu.VMEM((CH * L, D), jnp.float32),
        pltpu.VMEM((CH * L, D), jnp.float32),
        pltpu.VMEM((CH, D), jnp.float32),
        pltpu.VMEM((CH, D), jnp.float32),
        pltpu.VMEM((2, CH), jnp.float32),
        pltpu.VMEM((8, D), jnp.float32),
        pltpu.SemaphoreType.DMA,
        pltpu.SemaphoreType.DMA,
        pltpu.SemaphoreType.DMA,
        pltpu.SemaphoreType.DMA,
    ],
)(_text_body)

_id_call = functools.partial(
    pl.kernel,
    mesh=_MESH,
    compiler_params=_COMPILER_PARAMS,
    out_type=jax.ShapeDtypeStruct((B, 2 * D), jnp.float32),
    scratch_types=[
        pltpu.VMEM((CH,), jnp.int32),
        pltpu.VMEM((CH,), jnp.int32),
        pltpu.VMEM((CH, D), jnp.float32),
        pltpu.VMEM((CH, D), jnp.float32),
        pltpu.VMEM((CH, D), jnp.float32),
        pltpu.VMEM((CH, D), jnp.float32),
        pltpu.VMEM((CH, 2 * D), jnp.float32),
        pltpu.VMEM((CH, 2 * D), jnp.float32),
        pltpu.SemaphoreType.DMA,
        pltpu.SemaphoreType.DMA,
        pltpu.SemaphoreType.DMA,
        pltpu.SemaphoreType.DMA,
        pltpu.SemaphoreType.DMA,
        pltpu.SemaphoreType.DMA,
    ],
)(_id_body)


@jax.jit
def kernel(title_ids, token_ids, id_table, text_table):
    tok_flat = token_ids.reshape(B * L)
    text_emb = _text_call(tok_flat, text_table)
    return _id_call(title_ids, id_table, text_emb)


# speedup vs baseline: 1.0167x; 1.0167x over previous
import functools

import jax
import jax.numpy as jnp
from jax import lax
from jax.experimental import pallas as pl
from jax.experimental.pallas import tpu as pltpu
from jax.experimental.pallas import tpu_sc as plsc

B = 16384
L = 20
D = 32
NC = 2
NS = 16
NW = NC * NS
PER_W = B // NW
C = 128
NCHUNK = PER_W // C
TOKR = (C * L) // 128
LANES = 16

_COMPILER_PARAMS = pltpu.CompilerParams(use_tc_tiling_on_sc=False,
                                        needs_layout_passes=False)
_MESH = plsc.VectorSubcoreMesh(core_axis_name="c", subcore_axis_name="s")


CH = 64
NCH = PER_W // CH
CTOKR = (CH * L) // 128


def _text_body(tokr_hbm, text_hbm, out_hbm,
               tokidx_v, rows0_v, rows1_v, out0_v, out1_v, scale_v, row0_v,
               sem0, sem1, osem0, osem1):
    rows_v = (rows0_v, rows1_v)
    out_v = (out0_v, out1_v)
    sems = (sem0, sem1)
    osems = (osem0, osem1)
    wid = lax.axis_index("s") * NC + lax.axis_index("c")
    iota = lax.iota(jnp.int32, LANES)
    zv = jnp.zeros((LANES,), jnp.int32)
    ov = zv + 1

    wtok = PER_W * L
    pltpu.sync_copy(tokr_hbm.at[pl.ds(wid * wtok, wtok)], tokidx_v)
    pltpu.sync_copy(text_hbm.at[pl.ds(0, 8)], row0_v)

    def fire(c, buf):
        cps = []
        for r in range(CTOKR):
            cps.append(pltpu.async_copy(
                text_hbm.at[tokidx_v.at[pl.ds((c * CTOKR + r) * 128, 128)]],
                rows_v[buf].at[pl.ds(r * 128, 128)], sems[buf]))
        return cps

    r0a = row0_v[0, pl.ds(0, LANES)]
    r0b = row0_v[0, pl.ds(LANES, LANES)]
    inflight = {0: fire(0, 0)}
    owaits = {}

    for c in range(NCH):
        buf = c & 1
        base = wid * PER_W + c * CH

        if c + 1 < NCH:
            inflight[c + 1] = fire(c + 1, (c + 1) & 1)

        def zgroup(g, carry):
            brow = g * LANES + iota
            fb = brow * L + (c * CH * L)
            z = jnp.zeros((LANES,), jnp.float32)
            for l in range(L):
                tv = plsc.load_gather(tokidx_v, [fb + l])
                z = z + jnp.where(tv == 0, 1.0, 0.0)
            inv = 1.0 / jnp.maximum(L * 1.0 - z, 1.0)
            b0 = g * LANES
            scale_v[0, pl.ds(b0, LANES)] = inv
            scale_v[1, pl.ds(b0, LANES)] = z * inv
            return carry

        lax.fori_loop(0, CH // LANES, zgroup, 0)

        for cp in inflight.pop(c):
            cp.wait()
        if c - 2 in owaits:
            owaits.pop(c - 2).wait()

        def bloop(b, carry):
            fb = b * L
            a0 = rows_v[buf][fb, pl.ds(0, LANES)]
            a1 = rows_v[buf][fb, pl.ds(LANES, LANES)]
            for l in range(1, L):
                a0 = a0 + rows_v[buf][fb + l, pl.ds(0, LANES)]
                a1 = a1 + rows_v[buf][fb + l, pl.ds(LANES, LANES)]
            bcol = zv + b
            inv = plsc.load_gather(scale_v, [zv, bcol])
            zfi = plsc.load_gather(scale_v, [ov, bcol])
            out_v[buf][b, pl.ds(0, LANES)] = a0 * inv - zfi * r0a
            out_v[buf][b, pl.ds(LANES, LANES)] = a1 * inv - zfi * r0b
            return carry

        lax.fori_loop(0, CH, bloop, 0)
        owaits[c] = pltpu.async_copy(
            out_v[buf], out_hbm.at[pl.ds(base, CH)], osems[buf])
    for w in owaits.values():
        w.wait()


def _id_body(title_hbm, idt_hbm, text_hbm, out_hbm, ididx0_v, ididx1_v,
             idrows0_v, idrows1_v, textc0_v, textc1_v, out640_v, out641_v,
             sem0, sem1, tsem0, tsem1, osem0, osem1):
    ididx_v = (ididx0_v, ididx1_v)
    idrows_v = (idrows0_v, idrows1_v)
    textc_v = (textc0_v, textc1_v)
    out64_v = (out640_v, out641_v)
    sems = (sem0, sem1)
    tsems = (tsem0, tsem1)
    osems = (osem0, osem1)
    wid = lax.axis_index("s") * NC + lax.axis_index("c")

    def fire(c, buf):
        base = wid * PER_W + c * C
        pltpu.sync_copy(title_hbm.at[pl.ds(base, C)], ididx_v[buf])
        return (pltpu.async_copy(idt_hbm.at[ididx_v[buf]], idrows_v[buf],
                                 sems[buf]),
                pltpu.async_copy(text_hbm.at[pl.ds(base, C)], textc_v[buf],
                                 tsems[buf]))

    inflight = {0: fire(0, 0)}
    owaits = {}
    for c in range(NCHUNK):
        buf = c & 1
        base = wid * PER_W + c * C
        if c + 1 < NCHUNK:
            if c - 1 in owaits:
                owaits.pop(c - 1).wait()
            inflight[c + 1] = fire(c + 1, (c + 1) & 1)
        for cp in inflight.pop(c):
            cp.wait()

        def bloop(b, carry):
            out64_v[buf][b, pl.ds(0, LANES)] = idrows_v[buf][b, pl.ds(0, LANES)]
            out64_v[buf][b, pl.ds(LANES, LANES)] = (
                idrows_v[buf][b, pl.ds(LANES, LANES)])
            out64_v[buf][b, pl.ds(2 * LANES, LANES)] = (
                textc_v[buf][b, pl.ds(0, LANES)])
            out64_v[buf][b, pl.ds(3 * LANES, LANES)] = (
                textc_v[buf][b, pl.ds(LANES, LANES)])
            return carry

        lax.fori_loop(0, C, bloop, 0)
        owaits[c] = pltpu.async_copy(
            out64_v[buf], out_hbm.at[pl.ds(base, C)], osems[buf])
    for w in owaits.values():
        w.wait()


_text_call = functools.partial(
    pl.kernel,
    mesh=_MESH,
    compiler_params=_COMPILER_PARAMS,
    out_type=jax.ShapeDtypeStruct((B, D), jnp.float32),
    scratch_types=[
        pltpu.VMEM((PER_W * L,), jnp.int32),
        pltpu.VMEM((CH * L, D), jnp.float32),
        pltpu.VMEM((CH * L, D), jnp.float32),
        pltpu.VMEM((CH, D), jnp.float32),
        pltpu.VMEM((CH, D), jnp.float32),
        pltpu.VMEM((2, CH), jnp.float32),
        pltpu.VMEM((8, D), jnp.float32),
        pltpu.SemaphoreType.DMA,
        pltpu.SemaphoreType.DMA,
        pltpu.SemaphoreType.DMA,
        pltpu.SemaphoreType.DMA,
    ],
)(_text_body)

_id_call = functools.partial(
    pl.kernel,
    mesh=_MESH,
    compiler_params=_COMPILER_PARAMS,
    out_type=jax.ShapeDtypeStruct((B, 2 * D), jnp.float32),
    scratch_types=[
        pltpu.VMEM((C,), jnp.int32),
        pltpu.VMEM((C,), jnp.int32),
        pltpu.VMEM((C, D), jnp.float32),
        pltpu.VMEM((C, D), jnp.float32),
        pltpu.VMEM((C, D), jnp.float32),
        pltpu.VMEM((C, D), jnp.float32),
        pltpu.VMEM((C, 2 * D), jnp.float32),
        pltpu.VMEM((C, 2 * D), jnp.float32),
        pltpu.SemaphoreType.DMA,
        pltpu.SemaphoreType.DMA,
        pltpu.SemaphoreType.DMA,
        pltpu.SemaphoreType.DMA,
        pltpu.SemaphoreType.DMA,
        pltpu.SemaphoreType.DMA,
    ],
)(_id_body)


@jax.jit
def kernel(title_ids, token_ids, id_table, text_table):
    tok_flat = token_ids.reshape(B * L)
    text_emb = _text_call(tok_flat, text_table)
    return _id_call(title_ids, id_table, text_emb)


# speedup vs baseline: 1.0281x; 1.0112x over previous
import functools

import jax
import jax.numpy as jnp
from jax import lax
from jax.experimental import pallas as pl
from jax.experimental.pallas import tpu as pltpu
from jax.experimental.pallas import tpu_sc as plsc

B = 16384
L = 20
D = 32
NC = 2
NS = 16
NW = NC * NS
PER_W = B // NW
C = 128
NCHUNK = PER_W // C
TOKR = (C * L) // 128
LANES = 16

_COMPILER_PARAMS = pltpu.CompilerParams(use_tc_tiling_on_sc=False,
                                        needs_layout_passes=False)
_MESH = plsc.VectorSubcoreMesh(core_axis_name="c", subcore_axis_name="s")


CH = 64
NCH = PER_W // CH
CTOKR = (CH * L) // 128


def _text_body(tokr_hbm, text_hbm, out_hbm,
               tokidx_v, rows0_v, rows1_v, out0_v, out1_v, scale_v, row0_v,
               sem0, sem1, osem0, osem1):
    rows_v = (rows0_v, rows1_v)
    out_v = (out0_v, out1_v)
    sems = (sem0, sem1)
    osems = (osem0, osem1)
    wid = lax.axis_index("s") * NC + lax.axis_index("c")
    iota = lax.iota(jnp.int32, LANES)
    zv = jnp.zeros((LANES,), jnp.int32)
    ov = zv + 1

    wtok = PER_W * L
    pltpu.sync_copy(tokr_hbm.at[pl.ds(wid * wtok, wtok)], tokidx_v)
    pltpu.sync_copy(text_hbm.at[pl.ds(0, 8)], row0_v)

    def fire(c, buf):
        cps = []
        for r in range(CTOKR):
            cps.append(pltpu.async_copy(
                text_hbm.at[tokidx_v.at[pl.ds((c * CTOKR + r) * 128, 128)]],
                rows_v[buf].at[pl.ds(r * 128, 128)], sems[buf]))
        return cps

    r0a = row0_v[0, pl.ds(0, LANES)]
    r0b = row0_v[0, pl.ds(LANES, LANES)]
    inflight = {0: fire(0, 0)}
    owaits = {}

    for c in range(NCH):
        buf = c & 1
        base = wid * PER_W + c * CH

        if c + 1 < NCH:
            inflight[c + 1] = fire(c + 1, (c + 1) & 1)

        def zgroup(g, carry):
            brow = g * LANES + iota
            fb = brow * L + (c * CH * L)
            z = jnp.zeros((LANES,), jnp.float32)
            for l in range(L):
                tv = plsc.load_gather(tokidx_v, [fb + l])
                z = z + jnp.where(tv == 0, 1.0, 0.0)
            inv = 1.0 / jnp.maximum(L * 1.0 - z, 1.0)
            b0 = g * LANES
            scale_v[0, pl.ds(b0, LANES)] = inv
            scale_v[1, pl.ds(b0, LANES)] = z * inv
            return carry

        lax.fori_loop(0, CH // LANES, zgroup, 0)

        for cp in inflight.pop(c):
            cp.wait()
        if c - 2 in owaits:
            owaits.pop(c - 2).wait()

        def bloop(b, carry):
            fb = b * L
            a0 = rows_v[buf][fb, pl.ds(0, LANES)]
            a1 = rows_v[buf][fb, pl.ds(LANES, LANES)]
            for l in range(1, L):
                a0 = a0 + rows_v[buf][fb + l, pl.ds(0, LANES)]
                a1 = a1 + rows_v[buf][fb + l, pl.ds(LANES, LANES)]
            bcol = zv + b
            inv = plsc.load_gather(scale_v, [zv, bcol])
            zfi = plsc.load_gather(scale_v, [ov, bcol])
            out_v[buf][b, pl.ds(0, LANES)] = a0 * inv - zfi * r0a
            out_v[buf][b, pl.ds(LANES, LANES)] = a1 * inv - zfi * r0b
            return carry

        lax.fori_loop(0, CH, bloop, 0)
        owaits[c] = pltpu.async_copy(
            out_v[buf], out_hbm.at[pl.ds(base, CH)], osems[buf])
    for w in owaits.values():
        w.wait()


def _id_body(title_hbm, idt_hbm, text_hbm, out_hbm, ididx0_v, ididx1_v,
             idrows0_v, idrows1_v, textc0_v, textc1_v, out640_v, out641_v,
             sem0, sem1, tsem0, tsem1, osem0, osem1):
    ididx_v = (ididx0_v, ididx1_v)
    idrows_v = (idrows0_v, idrows1_v)
    textc_v = (textc0_v, textc1_v)
    out64_v = (out640_v, out641_v)
    sems = (sem0, sem1)
    tsems = (tsem0, tsem1)
    osems = (osem0, osem1)
    wid = lax.axis_index("s") * NC + lax.axis_index("c")

    def fire(c, buf):
        base = wid * PER_W + c * C
        pltpu.sync_copy(title_hbm.at[pl.ds(base, C)], ididx_v[buf])
        return (pltpu.async_copy(idt_hbm.at[ididx_v[buf]], idrows_v[buf],
                                 sems[buf]),
                pltpu.async_copy(text_hbm.at[pl.ds(base, C)],
                                 out64_v[buf].at[:, pl.ds(D, D)],
                                 tsems[buf]))

    inflight = {0: fire(0, 0)}
    owaits = {}
    for c in range(NCHUNK):
        buf = c & 1
        base = wid * PER_W + c * C
        if c + 1 < NCHUNK:
            if c - 1 in owaits:
                owaits.pop(c - 1).wait()
            inflight[c + 1] = fire(c + 1, (c + 1) & 1)
        for cp in inflight.pop(c):
            cp.wait()

        def bloop(b, carry):
            out64_v[buf][b, pl.ds(0, LANES)] = idrows_v[buf][b, pl.ds(0, LANES)]
            out64_v[buf][b, pl.ds(LANES, LANES)] = (
                idrows_v[buf][b, pl.ds(LANES, LANES)])
            return carry

        lax.fori_loop(0, C, bloop, 0)
        owaits[c] = pltpu.async_copy(
            out64_v[buf], out_hbm.at[pl.ds(base, C)], osems[buf])
    for w in owaits.values():
        w.wait()


_text_call = functools.partial(
    pl.kernel,
    mesh=_MESH,
    compiler_params=_COMPILER_PARAMS,
    out_type=jax.ShapeDtypeStruct((B, D), jnp.float32),
    scratch_types=[
        pltpu.VMEM((PER_W * L,), jnp.int32),
        pltpu.VMEM((CH * L, D), jnp.float32),
        pltpu.VMEM((CH * L, D), jnp.float32),
        pltpu.VMEM((CH, D), jnp.float32),
        pltpu.VMEM((CH, D), jnp.float32),
        pltpu.VMEM((2, CH), jnp.float32),
        pltpu.VMEM((8, D), jnp.float32),
        pltpu.SemaphoreType.DMA,
        pltpu.SemaphoreType.DMA,
        pltpu.SemaphoreType.DMA,
        pltpu.SemaphoreType.DMA,
    ],
)(_text_body)

_id_call = functools.partial(
    pl.kernel,
    mesh=_MESH,
    compiler_params=_COMPILER_PARAMS,
    out_type=jax.ShapeDtypeStruct((B, 2 * D), jnp.float32),
    scratch_types=[
        pltpu.VMEM((C,), jnp.int32),
        pltpu.VMEM((C,), jnp.int32),
        pltpu.VMEM((C, D), jnp.float32),
        pltpu.VMEM((C, D), jnp.float32),
        pltpu.VMEM((C, D), jnp.float32),
        pltpu.VMEM((C, D), jnp.float32),
        pltpu.VMEM((C, 2 * D), jnp.float32),
        pltpu.VMEM((C, 2 * D), jnp.float32),
        pltpu.SemaphoreType.DMA,
        pltpu.SemaphoreType.DMA,
        pltpu.SemaphoreType.DMA,
        pltpu.SemaphoreType.DMA,
        pltpu.SemaphoreType.DMA,
        pltpu.SemaphoreType.DMA,
    ],
)(_id_body)


@jax.jit
def kernel(title_ids, token_ids, id_table, text_table):
    tok_flat = token_ids.reshape(B * L)
    text_emb = _text_call(tok_flat, text_table)
    return _id_call(title_ids, id_table, text_emb)
